# Initial kernel scaffold; baseline (speedup 1.0000x reference)
import functools

import jax
import jax.numpy as jnp
from jax import lax
from jax.experimental import pallas as pl
from jax.experimental.pallas import tpu as pltpu
from jax.experimental.pallas import tpu_sc as plsc

N_ATOM = 10000
N_BOND = 10000
N_GLOBAL = 64
E_BOND = 320000
E_GLOBAL = 10000
D = 128
DW = D + 16
ACC_ROWS = 10016
NC, NS = 2, 16
NW = NC * NS
CHUNK = 128
ROWS_PER_SUB = ACC_ROWS // NS


def _ceil_div(a, b):
    return (a + b - 1) // b


@functools.lru_cache(maxsize=None)
def _make_segsum(n_chunks):
    mesh = plsc.VectorSubcoreMesh(core_axis_name="c", subcore_axis_name="s")

    @functools.partial(
        pl.kernel,
        out_type=jax.ShapeDtypeStruct((NC, ACC_ROWS, DW), jnp.float32),
        mesh=mesh,
        scratch_types=[
            pltpu.VMEM((n_chunks, CHUNK), jnp.int32),
            pltpu.VMEM((n_chunks, CHUNK), jnp.int32),
            pltpu.VMEM((CHUNK, DW), jnp.float32),
            pltpu.VMEM_SHARED((ACC_ROWS, DW), jnp.float32),
            pltpu.SemaphoreType.DMA,
        ],
    )
    def segsum(table_hbm, src_hbm, dst_hbm, zeros_hbm, out_hbm,
               src_v, dst_v, rows_v, acc, sem):
        cid = lax.axis_index("c")
        sid = lax.axis_index("s")
        wid = cid * NS + sid
        r0 = sid * ROWS_PER_SUB
        pltpu.sync_copy(zeros_hbm.at[pl.ds(r0, ROWS_PER_SUB)],
                        acc.at[pl.ds(r0, ROWS_PER_SUB)])
        pltpu.sync_copy(src_hbm.at[wid], src_v)
        pltpu.sync_copy(dst_hbm.at[wid], dst_v)
        plsc.subcore_barrier()

        @pl.loop(0, n_chunks)
        def _(j):
            pltpu.async_copy(table_hbm.at[src_v.at[j]], rows_v, sem).wait()
            pltpu.sync_copy(rows_v, acc.at[dst_v.at[j]], add=True)

        plsc.subcore_barrier()
        pltpu.sync_copy(acc.at[pl.ds(r0, ROWS_PER_SUB)],
                        out_hbm.at[cid].at[pl.ds(r0, ROWS_PER_SUB)])

    return segsum


def _edge_slabs(src, dst, n_chunks):
    e_pad = NW * n_chunks * CHUNK
    pad = e_pad - src.shape[0]
    src_p = jnp.concatenate([src, jnp.zeros((pad,), jnp.int32)])
    dst_p = jnp.concatenate([dst, jnp.full((pad,), N_ATOM, jnp.int32)])
    return (src_p.reshape(NW, n_chunks, CHUNK),
            dst_p.reshape(NW, n_chunks, CHUNK))


BLK = 1000


def _mlp_body(master_ref, p1_ref, p2_ref, w1_ref, b1_ref, w2_ref, b2_ref,
              w3_ref, b3_ref, out_ref):
    s1 = p1_ref[0] + p1_ref[1]
    s2 = p2_ref[0] + p2_ref[1]
    m1 = s1[:, :D] / jnp.maximum(s1[:, D:D + 1], 1.0)
    m2 = s2[:, :D] / jnp.maximum(s2[:, D:D + 1], 1.0)
    ft = jnp.concatenate([master_ref[...], m1, m2], axis=1)
    h = jax.nn.softplus(
        jnp.dot(ft, w1_ref[...], preferred_element_type=jnp.float32,
                precision=lax.Precision.HIGHEST) + b1_ref[...])
    h = jax.nn.softplus(
        jnp.dot(h, w2_ref[...], preferred_element_type=jnp.float32,
                precision=lax.Precision.HIGHEST) + b2_ref[...])
    out_ref[...] = (
        jnp.dot(h, w3_ref[...], preferred_element_type=jnp.float32,
                precision=lax.Precision.HIGHEST) + b3_ref[...])


def _mlp(master, p1, p2, W1, b1, W2, b2, W3, b3):
    n_blk = N_ATOM // BLK
    return pl.pallas_call(
        _mlp_body,
        grid=(n_blk,),
        in_specs=[
            pl.BlockSpec((BLK, D), lambda i: (i, 0)),
            pl.BlockSpec((NC, BLK, DW), lambda i: (0, i, 0)),
            pl.BlockSpec((NC, BLK, DW), lambda i: (0, i, 0)),
            pl.BlockSpec((3 * D, 64), lambda i: (0, 0)),
            pl.BlockSpec((1, 64), lambda i: (0, 0)),
            pl.BlockSpec((64, 64), lambda i: (0, 0)),
            pl.BlockSpec((1, 64), lambda i: (0, 0)),
            pl.BlockSpec((64, 32), lambda i: (0, 0)),
            pl.BlockSpec((1, 32), lambda i: (0, 0)),
        ],
        out_specs=pl.BlockSpec((BLK, 32), lambda i: (i, 0)),
        out_shape=jax.ShapeDtypeStruct((N_ATOM, 32), jnp.float32),
    )(master, p1, p2, W1, b1.reshape(1, -1), W2, b2.reshape(1, -1),
      W3, b3.reshape(1, -1))


def kernel(master_feats, bond_feats, global_feats, edge_index_bond,
           src_global, dst_global, W1, b1, W2, b2, W3, b3):
    ones16_b = jnp.ones((N_BOND, 16), jnp.float32)
    ones16_g = jnp.ones((N_GLOBAL, 16), jnp.float32)
    bond_ext = jnp.concatenate([bond_feats, ones16_b], axis=1)
    glob_ext = jnp.concatenate([global_feats, ones16_g], axis=1)
    zeros = jnp.zeros((ACC_ROWS, DW), jnp.float32)

    nch1 = _ceil_div(E_BOND, NW * CHUNK)
    nch2 = _ceil_div(E_GLOBAL, NW * CHUNK)
    src1, dst1 = _edge_slabs(edge_index_bond[0], edge_index_bond[1], nch1)
    src2, dst2 = _edge_slabs(src_global, dst_global, nch2)

    p1 = _make_segsum(nch1)(bond_ext, src1, dst1, zeros)
    p2 = _make_segsum(nch2)(glob_ext, src2, dst2, zeros)

    return _mlp(master_feats, p1, p2, W1, b1, W2, b2, W3, b3)


# speedup vs baseline: 3.1585x; 3.1585x over previous
import functools

import jax
import jax.numpy as jnp
from jax import lax
from jax.experimental import pallas as pl
from jax.experimental.pallas import tpu as pltpu
from jax.experimental.pallas import tpu_sc as plsc

N_ATOM = 10000
N_BOND = 10000
N_GLOBAL = 64
E_BOND = 320000
E_GLOBAL = 10000
D = 128
DW = D + 16
ACC_ROWS = 10112
NC, NS = 2, 16
NW = NC * NS
CHUNK = 128
ROWS_PER_SUB = ACC_ROWS // NS


def _ceil_div(a, b):
    return (a + b - 1) // b


@functools.lru_cache(maxsize=None)
def _make_segsum(n_chunks):
    mesh = plsc.VectorSubcoreMesh(core_axis_name="c", subcore_axis_name="s")

    @functools.partial(
        pl.kernel,
        out_type=jax.ShapeDtypeStruct((NC, ACC_ROWS, DW), jnp.float32),
        mesh=mesh,
        scratch_types=[
            pltpu.VMEM((n_chunks, CHUNK), jnp.int32),
            pltpu.VMEM((n_chunks, CHUNK), jnp.int32),
            pltpu.VMEM((CHUNK, DW), jnp.float32),
            pltpu.VMEM_SHARED((ACC_ROWS, DW), jnp.float32),
            pltpu.SemaphoreType.DMA,
        ],
        compiler_params=pltpu.CompilerParams(use_tc_tiling_on_sc=False),
    )
    def segsum(table_hbm, src_hbm, dst_hbm, zeros_hbm, out_hbm,
               src_v, dst_v, rows_v, acc, sem):
        cid = lax.axis_index("c")
        sid = lax.axis_index("s")
        wid = cid * NS + sid
        r0 = sid * ROWS_PER_SUB
        pltpu.sync_copy(zeros_hbm.at[pl.ds(r0, ROWS_PER_SUB)],
                        acc.at[pl.ds(r0, ROWS_PER_SUB)])
        pltpu.sync_copy(src_hbm.at[wid], src_v)
        pltpu.sync_copy(dst_hbm.at[wid], dst_v)
        plsc.subcore_barrier()

        @pl.loop(0, n_chunks)
        def _(j):
            pltpu.async_copy(table_hbm.at[src_v.at[j]], rows_v, sem).wait()
            pltpu.sync_copy(rows_v, acc.at[dst_v.at[j]], add=True)

        plsc.subcore_barrier()
        pltpu.sync_copy(acc.at[pl.ds(r0, ROWS_PER_SUB)],
                        out_hbm.at[cid].at[pl.ds(r0, ROWS_PER_SUB)])

    return segsum


def _edge_slabs(src, dst, n_chunks):
    e_pad = NW * n_chunks * CHUNK
    pad = e_pad - src.shape[0]
    src_p = jnp.concatenate([src, jnp.zeros((pad,), jnp.int32)])
    dst_p = jnp.concatenate([dst, jnp.full((pad,), N_ATOM, jnp.int32)])
    return (src_p.reshape(NW, n_chunks, CHUNK),
            dst_p.reshape(NW, n_chunks, CHUNK))


BLK = 1000


def _mlp_body(master_ref, p1_ref, p2_ref, w1_ref, b1_ref, w2_ref, b2_ref,
              w3_ref, b3_ref, out_ref):
    s1 = p1_ref[0] + p1_ref[1]
    s2 = p2_ref[0] + p2_ref[1]
    m1 = s1[:, :D] / jnp.maximum(s1[:, D:D + 1], 1.0)
    m2 = s2[:, :D] / jnp.maximum(s2[:, D:D + 1], 1.0)
    ft = jnp.concatenate([master_ref[...], m1, m2], axis=1)
    h = jax.nn.softplus(
        jnp.dot(ft, w1_ref[...], preferred_element_type=jnp.float32,
                precision=lax.Precision.HIGHEST) + b1_ref[...])
    h = jax.nn.softplus(
        jnp.dot(h, w2_ref[...], preferred_element_type=jnp.float32,
                precision=lax.Precision.HIGHEST) + b2_ref[...])
    out_ref[...] = (
        jnp.dot(h, w3_ref[...], preferred_element_type=jnp.float32,
                precision=lax.Precision.HIGHEST) + b3_ref[...])


def _mlp(master, p1, p2, W1, b1, W2, b2, W3, b3):
    n_blk = N_ATOM // BLK
    return pl.pallas_call(
        _mlp_body,
        grid=(n_blk,),
        in_specs=[
            pl.BlockSpec((BLK, D), lambda i: (i, 0)),
            pl.BlockSpec((NC, BLK, DW), lambda i: (0, i, 0)),
            pl.BlockSpec((NC, BLK, DW), lambda i: (0, i, 0)),
            pl.BlockSpec((3 * D, 64), lambda i: (0, 0)),
            pl.BlockSpec((1, 64), lambda i: (0, 0)),
            pl.BlockSpec((64, 64), lambda i: (0, 0)),
            pl.BlockSpec((1, 64), lambda i: (0, 0)),
            pl.BlockSpec((64, 32), lambda i: (0, 0)),
            pl.BlockSpec((1, 32), lambda i: (0, 0)),
        ],
        out_specs=pl.BlockSpec((BLK, 32), lambda i: (i, 0)),
        out_shape=jax.ShapeDtypeStruct((N_ATOM, 32), jnp.float32),
    )(master, p1, p2, W1, b1.reshape(1, -1), W2, b2.reshape(1, -1),
      W3, b3.reshape(1, -1))


def kernel(master_feats, bond_feats, global_feats, edge_index_bond,
           src_global, dst_global, W1, b1, W2, b2, W3, b3):
    ones16_b = jnp.ones((N_BOND, 16), jnp.float32)
    ones16_g = jnp.ones((N_GLOBAL, 16), jnp.float32)
    bond_ext = jnp.concatenate([bond_feats, ones16_b], axis=1)
    glob_ext = jnp.concatenate([global_feats, ones16_g], axis=1)
    zeros = jnp.zeros((ACC_ROWS, DW), jnp.float32)

    nch1 = _ceil_div(E_BOND, NW * CHUNK)
    nch2 = _ceil_div(E_GLOBAL, NW * CHUNK)
    src1, dst1 = _edge_slabs(edge_index_bond[0], edge_index_bond[1], nch1)
    src2, dst2 = _edge_slabs(src_global, dst_global, nch2)

    p1 = _make_segsum(nch1)(bond_ext, src1, dst1, zeros)
    p2 = _make_segsum(nch2)(glob_ext, src2, dst2, zeros)

    return _mlp(master_feats, p1, p2, W1, b1, W2, b2, W3, b3)


# speedup vs baseline: 3.4954x; 1.1067x over previous
import functools

import jax
import jax.numpy as jnp
from jax import lax
from jax.experimental import pallas as pl
from jax.experimental.pallas import tpu as pltpu
from jax.experimental.pallas import tpu_sc as plsc

N_ATOM = 10000
N_BOND = 10000
N_GLOBAL = 64
E_BOND = 320000
E_GLOBAL = 10000
D = 128
DW = D + 16
ACC_ROWS = 10112
NC, NS = 2, 16
NW = NC * NS
CHUNK = 64
ROWS_PER_SUB = ACC_ROWS // NS


def _ceil_div(a, b):
    return (a + b - 1) // b


def _even_chunks(n_edges):
    n = _ceil_div(n_edges, NW * CHUNK)
    return n + (n % 2)


@functools.lru_cache(maxsize=None)
def _make_segsum(n_chunks):
    mesh = plsc.VectorSubcoreMesh(core_axis_name="c", subcore_axis_name="s")

    @functools.partial(
        pl.kernel,
        out_type=jax.ShapeDtypeStruct((NC, ACC_ROWS, DW), jnp.float32),
        mesh=mesh,
        scratch_types=[
            pltpu.VMEM((n_chunks, CHUNK), jnp.int32),
            pltpu.VMEM((n_chunks, CHUNK), jnp.int32),
            pltpu.VMEM((CHUNK, DW), jnp.float32),
            pltpu.VMEM((CHUNK, DW), jnp.float32),
            pltpu.VMEM_SHARED((ACC_ROWS, DW), jnp.float32),
            pltpu.SemaphoreType.DMA,
            pltpu.SemaphoreType.DMA,
        ],
        compiler_params=pltpu.CompilerParams(use_tc_tiling_on_sc=False),
    )
    def segsum(table_hbm, src_hbm, dst_hbm, zeros_hbm, out_hbm,
               src_v, dst_v, rows_a, rows_b, acc, sem_a, sem_b):
        cid = lax.axis_index("c")
        sid = lax.axis_index("s")
        wid = cid * NS + sid
        r0 = sid * ROWS_PER_SUB
        pltpu.sync_copy(zeros_hbm.at[pl.ds(r0, ROWS_PER_SUB)],
                        acc.at[pl.ds(r0, ROWS_PER_SUB)])
        pltpu.sync_copy(src_hbm.at[wid], src_v)
        pltpu.sync_copy(dst_hbm.at[wid], dst_v)
        plsc.subcore_barrier()

        pltpu.async_copy(table_hbm.at[src_v.at[0]], rows_a, sem_a)

        @pl.loop(0, n_chunks, step=2)
        def _(j):
            pltpu.async_copy(table_hbm.at[src_v.at[j + 1]], rows_b, sem_b)
            pltpu.make_async_copy(table_hbm.at[src_v.at[j]], rows_a,
                                  sem_a).wait()
            pltpu.sync_copy(rows_a, acc.at[dst_v.at[j]], add=True)

            @pl.when(j + 2 < n_chunks)
            def _():
                pltpu.async_copy(table_hbm.at[src_v.at[j + 2]], rows_a, sem_a)

            pltpu.make_async_copy(table_hbm.at[src_v.at[j + 1]], rows_b,
                                  sem_b).wait()
            pltpu.sync_copy(rows_b, acc.at[dst_v.at[j + 1]], add=True)

        plsc.subcore_barrier()
        pltpu.sync_copy(acc.at[pl.ds(r0, ROWS_PER_SUB)],
                        out_hbm.at[cid].at[pl.ds(r0, ROWS_PER_SUB)])

    return segsum


def _edge_slabs(src, dst, n_chunks):
    e_pad = NW * n_chunks * CHUNK
    pad = e_pad - src.shape[0]
    src_p = jnp.concatenate([src, jnp.zeros((pad,), jnp.int32)])
    dst_p = jnp.concatenate([dst, jnp.full((pad,), N_ATOM, jnp.int32)])
    return (src_p.reshape(NW, n_chunks, CHUNK),
            dst_p.reshape(NW, n_chunks, CHUNK))


BLK = 1000


def _mlp_body(master_ref, p1_ref, p2_ref, w1_ref, b1_ref, w2_ref, b2_ref,
              w3_ref, b3_ref, out_ref):
    s1 = p1_ref[0] + p1_ref[1]
    s2 = p2_ref[0] + p2_ref[1]
    m1 = s1[:, :D] / jnp.maximum(s1[:, D:D + 1], 1.0)
    m2 = s2[:, :D] / jnp.maximum(s2[:, D:D + 1], 1.0)
    ft = jnp.concatenate([master_ref[...], m1, m2], axis=1)
    h = jax.nn.softplus(
        jnp.dot(ft, w1_ref[...], preferred_element_type=jnp.float32,
                precision=lax.Precision.HIGHEST) + b1_ref[...])
    h = jax.nn.softplus(
        jnp.dot(h, w2_ref[...], preferred_element_type=jnp.float32,
                precision=lax.Precision.HIGHEST) + b2_ref[...])
    out_ref[...] = (
        jnp.dot(h, w3_ref[...], preferred_element_type=jnp.float32,
                precision=lax.Precision.HIGHEST) + b3_ref[...])


def _mlp(master, p1, p2, W1, b1, W2, b2, W3, b3):
    n_blk = N_ATOM // BLK
    return pl.pallas_call(
        _mlp_body,
        grid=(n_blk,),
        in_specs=[
            pl.BlockSpec((BLK, D), lambda i: (i, 0)),
            pl.BlockSpec((NC, BLK, DW), lambda i: (0, i, 0)),
            pl.BlockSpec((NC, BLK, DW), lambda i: (0, i, 0)),
            pl.BlockSpec((3 * D, 64), lambda i: (0, 0)),
            pl.BlockSpec((1, 64), lambda i: (0, 0)),
            pl.BlockSpec((64, 64), lambda i: (0, 0)),
            pl.BlockSpec((1, 64), lambda i: (0, 0)),
            pl.BlockSpec((64, 32), lambda i: (0, 0)),
            pl.BlockSpec((1, 32), lambda i: (0, 0)),
        ],
        out_specs=pl.BlockSpec((BLK, 32), lambda i: (i, 0)),
        out_shape=jax.ShapeDtypeStruct((N_ATOM, 32), jnp.float32),
    )(master, p1, p2, W1, b1.reshape(1, -1), W2, b2.reshape(1, -1),
      W3, b3.reshape(1, -1))


def kernel(master_feats, bond_feats, global_feats, edge_index_bond,
           src_global, dst_global, W1, b1, W2, b2, W3, b3):
    ones16_b = jnp.ones((N_BOND, 16), jnp.float32)
    ones16_g = jnp.ones((N_GLOBAL, 16), jnp.float32)
    bond_ext = jnp.concatenate([bond_feats, ones16_b], axis=1)
    glob_ext = jnp.concatenate([global_feats, ones16_g], axis=1)
    zeros = jnp.zeros((ACC_ROWS, DW), jnp.float32)

    nch1 = _even_chunks(E_BOND)
    nch2 = _even_chunks(E_GLOBAL)
    src1, dst1 = _edge_slabs(edge_index_bond[0], edge_index_bond[1], nch1)
    src2, dst2 = _edge_slabs(src_global, dst_global, nch2)

    p1 = _make_segsum(nch1)(bond_ext, src1, dst1, zeros)
    p2 = _make_segsum(nch2)(glob_ext, src2, dst2, zeros)

    return _mlp(master_feats, p1, p2, W1, b1, W2, b2, W3, b3)


# speedup vs baseline: 3.5132x; 1.0051x over previous
import functools

import jax
import jax.numpy as jnp
from jax import lax
from jax.experimental import pallas as pl
from jax.experimental.pallas import tpu as pltpu
from jax.experimental.pallas import tpu_sc as plsc

N_ATOM = 10000
N_BOND = 10000
N_GLOBAL = 64
E_BOND = 320000
E_GLOBAL = 10000
D = 128
DW = D + 16
ACC_ROWS = 10112
NC, NS = 2, 16
NW = NC * NS
CHUNK = 64
ROWS_PER_SUB = ACC_ROWS // NS


def _ceil_div(a, b):
    return (a + b - 1) // b


def _even_chunks(n_edges):
    n = _ceil_div(n_edges, NW * CHUNK)
    return n + (n % 2)


@functools.lru_cache(maxsize=None)
def _make_segsum(nch1, nch2):
    mesh = plsc.VectorSubcoreMesh(core_axis_name="c", subcore_axis_name="s")

    @functools.partial(
        pl.kernel,
        out_type=(jax.ShapeDtypeStruct((NC, ACC_ROWS, DW), jnp.float32),
                  jax.ShapeDtypeStruct((NC, ACC_ROWS, DW), jnp.float32)),
        mesh=mesh,
        scratch_types=[
            pltpu.VMEM((nch1, CHUNK), jnp.int32),
            pltpu.VMEM((nch1, CHUNK), jnp.int32),
            pltpu.VMEM((CHUNK, DW), jnp.float32),
            pltpu.VMEM((CHUNK, DW), jnp.float32),
            pltpu.VMEM_SHARED((ACC_ROWS, DW), jnp.float32),
            pltpu.SemaphoreType.DMA,
            pltpu.SemaphoreType.DMA,
        ],
        compiler_params=pltpu.CompilerParams(use_tc_tiling_on_sc=False),
    )
    def segsum(tab1_hbm, tab2_hbm, src1_hbm, dst1_hbm, src2_hbm, dst2_hbm,
               zeros_hbm, out1_hbm, out2_hbm,
               src_v, dst_v, rows_a, rows_b, acc, sem_a, sem_b):
        cid = lax.axis_index("c")
        sid = lax.axis_index("s")
        wid = cid * NS + sid
        r0 = sid * ROWS_PER_SUB

        def phase(table_hbm, src_hbm, dst_hbm, out_hbm, n_chunks):
            pltpu.sync_copy(zeros_hbm.at[pl.ds(r0, ROWS_PER_SUB)],
                            acc.at[pl.ds(r0, ROWS_PER_SUB)])
            pltpu.sync_copy(src_hbm.at[wid], src_v.at[pl.ds(0, n_chunks)])
            pltpu.sync_copy(dst_hbm.at[wid], dst_v.at[pl.ds(0, n_chunks)])
            plsc.subcore_barrier()

            pltpu.async_copy(table_hbm.at[src_v.at[0]], rows_a, sem_a)

            @pl.loop(0, n_chunks, step=2)
            def _(j):
                pltpu.async_copy(table_hbm.at[src_v.at[j + 1]], rows_b, sem_b)
                pltpu.make_async_copy(table_hbm.at[src_v.at[j]], rows_a,
                                      sem_a).wait()
                pltpu.sync_copy(rows_a, acc.at[dst_v.at[j]], add=True)

                @pl.when(j + 2 < n_chunks)
                def _():
                    pltpu.async_copy(table_hbm.at[src_v.at[j + 2]], rows_a,
                                     sem_a)

                pltpu.make_async_copy(table_hbm.at[src_v.at[j + 1]], rows_b,
                                      sem_b).wait()
                pltpu.sync_copy(rows_b, acc.at[dst_v.at[j + 1]], add=True)

            plsc.subcore_barrier()
            pltpu.sync_copy(acc.at[pl.ds(r0, ROWS_PER_SUB)],
                            out_hbm.at[cid].at[pl.ds(r0, ROWS_PER_SUB)])

        phase(tab1_hbm, src1_hbm, dst1_hbm, out1_hbm, nch1)
        phase(tab2_hbm, src2_hbm, dst2_hbm, out2_hbm, nch2)

    return segsum


def _edge_slabs(src, dst, n_chunks):
    e_pad = NW * n_chunks * CHUNK
    pad = e_pad - src.shape[0]
    src_p = jnp.concatenate([src, jnp.zeros((pad,), jnp.int32)])
    dst_p = jnp.concatenate([dst, jnp.full((pad,), N_ATOM, jnp.int32)])
    return (src_p.reshape(NW, n_chunks, CHUNK),
            dst_p.reshape(NW, n_chunks, CHUNK))


BLK = 1000


def _mlp_body(master_ref, p1_ref, p2_ref, w1_ref, b1_ref, w2_ref, b2_ref,
              w3_ref, b3_ref, out_ref):
    s1 = p1_ref[0] + p1_ref[1]
    s2 = p2_ref[0] + p2_ref[1]
    m1 = s1[:, :D] / jnp.maximum(s1[:, D:D + 1], 1.0)
    m2 = s2[:, :D] / jnp.maximum(s2[:, D:D + 1], 1.0)
    ft = jnp.concatenate([master_ref[...], m1, m2], axis=1)
    h = jax.nn.softplus(
        jnp.dot(ft, w1_ref[...], preferred_element_type=jnp.float32,
                precision=lax.Precision.HIGHEST) + b1_ref[...])
    h = jax.nn.softplus(
        jnp.dot(h, w2_ref[...], preferred_element_type=jnp.float32,
                precision=lax.Precision.HIGHEST) + b2_ref[...])
    out_ref[...] = (
        jnp.dot(h, w3_ref[...], preferred_element_type=jnp.float32,
                precision=lax.Precision.HIGHEST) + b3_ref[...])


def _mlp(master, p1, p2, W1, b1, W2, b2, W3, b3):
    n_blk = N_ATOM // BLK
    return pl.pallas_call(
        _mlp_body,
        grid=(n_blk,),
        in_specs=[
            pl.BlockSpec((BLK, D), lambda i: (i, 0)),
            pl.BlockSpec((NC, BLK, DW), lambda i: (0, i, 0)),
            pl.BlockSpec((NC, BLK, DW), lambda i: (0, i, 0)),
            pl.BlockSpec((3 * D, 64), lambda i: (0, 0)),
            pl.BlockSpec((1, 64), lambda i: (0, 0)),
            pl.BlockSpec((64, 64), lambda i: (0, 0)),
            pl.BlockSpec((1, 64), lambda i: (0, 0)),
            pl.BlockSpec((64, 32), lambda i: (0, 0)),
            pl.BlockSpec((1, 32), lambda i: (0, 0)),
        ],
        out_specs=pl.BlockSpec((BLK, 32), lambda i: (i, 0)),
        out_shape=jax.ShapeDtypeStruct((N_ATOM, 32), jnp.float32),
    )(master, p1, p2, W1, b1.reshape(1, -1), W2, b2.reshape(1, -1),
      W3, b3.reshape(1, -1))


def kernel(master_feats, bond_feats, global_feats, edge_index_bond,
           src_global, dst_global, W1, b1, W2, b2, W3, b3):
    ones16_b = jnp.ones((N_BOND, 16), jnp.float32)
    ones16_g = jnp.ones((N_GLOBAL, 16), jnp.float32)
    bond_ext = jnp.concatenate([bond_feats, ones16_b], axis=1)
    glob_ext = jnp.concatenate([global_feats, ones16_g], axis=1)
    zeros = jnp.zeros((ACC_ROWS, DW), jnp.float32)

    nch1 = _even_chunks(E_BOND)
    nch2 = _even_chunks(E_GLOBAL)
    src1, dst1 = _edge_slabs(edge_index_bond[0], edge_index_bond[1], nch1)
    src2, dst2 = _edge_slabs(src_global, dst_global, nch2)

    p1, p2 = _make_segsum(nch1, nch2)(bond_ext, glob_ext, src1, dst1,
                                      src2, dst2, zeros)

    return _mlp(master_feats, p1, p2, W1, b1, W2, b2, W3, b3)
